# baseline (speedup 1.0000x reference)
import functools

import jax
import jax.numpy as jnp
from jax import lax
from jax.experimental import pallas as pl
from jax.experimental.pallas import tpu as pltpu
from jax.experimental.pallas import tpu_sc as plsc

T = 2048
DIM = 1024
INTER = 512
E = 64
K = 2
TK = T * K
TILE = 128
NT = 96
NP = NT * TILE

SC_CORES = 2
SC_SUBCORES = 16
NW = SC_CORES * SC_SUBCORES


def _router_body(x_ref, gwt_ref, eid_ref, g_ref):
    xb = x_ref[...]
    logits = jnp.dot(xb, gwt_ref[...], preferred_element_type=jnp.float32)
    scores = jax.nn.sigmoid(logits)
    cols = lax.broadcasted_iota(jnp.int32, scores.shape, 1)
    m1 = jnp.max(scores, axis=1)
    a1 = jnp.argmax(scores, axis=1).astype(jnp.int32)
    masked = jnp.where(cols == a1[:, None], -jnp.inf, scores)
    m2 = jnp.max(masked, axis=1)
    a2 = jnp.argmax(masked, axis=1).astype(jnp.int32)
    s = jnp.maximum(m1 + m2, 1e-12)
    eid_ref[...] = jnp.concatenate([a1[:, None], a2[:, None]], axis=1)
    g_ref[...] = jnp.concatenate([(m1 / s)[:, None], (m2 / s)[:, None]], axis=1)


def _router(x, gwt):
    bt = 512
    grid = (T // bt,)
    return pl.pallas_call(
        _router_body,
        grid=grid,
        in_specs=[
            pl.BlockSpec((bt, DIM), lambda i: (i, 0)),
            pl.BlockSpec((DIM, E), lambda i: (0, 0)),
        ],
        out_specs=[
            pl.BlockSpec((bt, K), lambda i: (i, 0)),
            pl.BlockSpec((bt, K), lambda i: (i, 0)),
        ],
        out_shape=[
            jax.ShapeDtypeStruct((T, K), jnp.int32),
            jax.ShapeDtypeStruct((T, K), jnp.float32),
        ],
    )(x, gwt)


def _shared_body(x_ref, s1_ref, s3_ref, s2_ref, sh_ref):
    xb = x_ref[...]
    h = jax.nn.silu(jnp.dot(xb, s1_ref[...], preferred_element_type=jnp.float32))
    h = h * jnp.dot(xb, s3_ref[...], preferred_element_type=jnp.float32)
    sh_ref[...] = jnp.dot(h, s2_ref[...], preferred_element_type=jnp.float32)


def _shared(x, s1, s3, s2):
    bt = 256
    grid = (T // bt,)
    return pl.pallas_call(
        _shared_body,
        grid=grid,
        in_specs=[
            pl.BlockSpec((bt, DIM), lambda i: (i, 0)),
            pl.BlockSpec((DIM, DIM), lambda i: (0, 0)),
            pl.BlockSpec((DIM, DIM), lambda i: (0, 0)),
            pl.BlockSpec((DIM, DIM), lambda i: (0, 0)),
        ],
        out_specs=pl.BlockSpec((bt, DIM), lambda i: (i, 0)),
        out_shape=jax.ShapeDtypeStruct((T, DIM), jnp.float32),
    )(x, s1, s3, s2)


def _gemm_body(texp_ref, act_ref, gtok_ref, x_any, w1_ref, w3_ref, w2_ref,
               gw_ref, ys_ref, rows, sems):
    j = pl.program_id(0)

    def issue(tj):
        slot = lax.rem(tj, 2)
        base = tj * TILE

        def cp(i, c):
            tok = jnp.clip(gtok_ref[base + i], 0, T - 1)
            pltpu.make_async_copy(
                x_any.at[pl.ds(tok, 1), :],
                rows.at[slot, pl.ds(i, 1), :],
                sems.at[slot],
            ).start()
            return c

        lax.fori_loop(0, TILE, cp, 0)

    @pl.when(j == 0)
    def _():
        issue(0)

    nxt = jnp.minimum(j + 1, NT - 1)

    @pl.when(jnp.logical_and(j + 1 < NT, act_ref[nxt] != 0))
    def _():
        issue(j + 1)

    @pl.when(act_ref[j] != 0)
    def _():
        slot = lax.rem(j, 2)
        pltpu.make_async_copy(
            x_any.at[pl.ds(0, TILE), :], rows.at[slot], sems.at[slot],
        ).wait()
        xb = rows[slot]
        h = jax.nn.silu(jnp.dot(xb, w1_ref[0], preferred_element_type=jnp.float32))
        h = h * jnp.dot(xb, w3_ref[0], preferred_element_type=jnp.float32)
        y = jnp.dot(h, w2_ref[0], preferred_element_type=jnp.float32)
        ys_ref[...] = y * gw_ref[0, 0][:, None]


def _grouped_gemm(texp, act, gtok, x, W1, W3, W2, gw3):
    grid_spec = pltpu.PrefetchScalarGridSpec(
        num_scalar_prefetch=3,
        grid=(NT,),
        in_specs=[
            pl.BlockSpec(memory_space=pl.ANY),
            pl.BlockSpec((1, DIM, INTER), lambda j, texp, act, gtok: (texp[j], 0, 0)),
            pl.BlockSpec((1, DIM, INTER), lambda j, texp, act, gtok: (texp[j], 0, 0)),
            pl.BlockSpec((1, INTER, DIM), lambda j, texp, act, gtok: (texp[j], 0, 0)),
            pl.BlockSpec((1, 1, TILE), lambda j, texp, act, gtok: (j, 0, 0)),
        ],
        out_specs=pl.BlockSpec((TILE, DIM), lambda j, texp, act, gtok: (j, 0)),
        scratch_shapes=[
            pltpu.VMEM((2, TILE, DIM), jnp.float32),
            pltpu.SemaphoreType.DMA((2,)),
        ],
    )
    return pl.pallas_call(
        _gemm_body,
        grid_spec=grid_spec,
        out_shape=jax.ShapeDtypeStruct((NP, DIM), jnp.float32),
    )(texp, act, gtok, x, W1, W3, W2, gw3)


def _sc_scatter(pos, tokv, wv):
    ppw = TK // NW
    mesh = plsc.VectorSubcoreMesh(core_axis_name="c", subcore_axis_name="s")

    @functools.partial(
        pl.kernel,
        mesh=mesh,
        out_type=[
            jax.ShapeDtypeStruct((NP,), jnp.int32),
            jax.ShapeDtypeStruct((NP,), jnp.float32),
        ],
        scratch_types=[
            pltpu.VMEM((ppw,), jnp.int32),
            pltpu.VMEM((ppw,), jnp.int32),
            pltpu.VMEM((ppw,), jnp.float32),
            pltpu.SemaphoreType.DMA,
            pltpu.SemaphoreType.DMA,
        ],
    )
    def k(pos_hbm, tok_hbm, wv_hbm, gtok_hbm, gwf_hbm, idx_v, tv, wvv, s1, s2):
        wid = lax.axis_index("s") * SC_CORES + lax.axis_index("c")
        base = pl.multiple_of(wid * ppw, ppw)
        pltpu.sync_copy(pos_hbm.at[pl.ds(base, ppw)], idx_v)
        pltpu.sync_copy(tok_hbm.at[pl.ds(base, ppw)], tv)
        pltpu.sync_copy(wv_hbm.at[pl.ds(base, ppw)], wvv)
        h1 = pltpu.async_copy(tv, gtok_hbm.at[idx_v], s1)
        h2 = pltpu.async_copy(wvv, gwf_hbm.at[idx_v], s2)
        h1.wait()
        h2.wait()

    return k(pos, tokv, wv)


def _sc_combine(ys, sh, pp0, pp1):
    tok_pw = T // NW
    ch = 32
    mesh = plsc.VectorSubcoreMesh(core_axis_name="c", subcore_axis_name="s")

    @functools.partial(
        pl.kernel,
        mesh=mesh,
        out_type=jax.ShapeDtypeStruct((T, DIM), jnp.float32),
        scratch_types=[
            pltpu.VMEM((ch,), jnp.int32),
            pltpu.VMEM((ch,), jnp.int32),
            pltpu.VMEM((ch, DIM), jnp.float32),
            pltpu.VMEM((ch, DIM), jnp.float32),
            pltpu.VMEM((ch, DIM), jnp.float32),
            pltpu.SemaphoreType.DMA,
        ],
    )
    def k(ys_hbm, sh_hbm, pp0_hbm, pp1_hbm, out_hbm, i0v, i1v, b0, b1, bs,
          sem):
        wid = lax.axis_index("s") * SC_CORES + lax.axis_index("c")
        base = wid * tok_pw

        def chunk(c, carry):
            off = pl.multiple_of(base + c * ch, ch)
            pltpu.sync_copy(pp0_hbm.at[pl.ds(off, ch)], i0v)
            pltpu.sync_copy(pp1_hbm.at[pl.ds(off, ch)], i1v)
            pltpu.async_copy(ys_hbm.at[i0v], b0, sem).wait()
            pltpu.async_copy(ys_hbm.at[i1v], b1, sem).wait()
            pltpu.sync_copy(sh_hbm.at[pl.ds(off, ch)], bs)

            def row(r, rc):
                def col(cc, cc2):
                    sl = pl.ds(pl.multiple_of(cc * 16, 16), 16)
                    b0[r, sl] = b0[r, sl] + b1[r, sl] + bs[r, sl]
                    return cc2
                lax.fori_loop(0, DIM // 16, col, 0)
                return rc

            lax.fori_loop(0, ch, row, 0)
            pltpu.sync_copy(b0, out_hbm.at[pl.ds(off, ch)])
            return carry

        lax.fori_loop(0, tok_pw // ch, chunk, 0)

    return k(ys, sh, pp0, pp1)


def _routing_metadata(eid, g):
    ef = eid.reshape(-1)
    gf = g.reshape(-1)
    onehot = (ef[:, None] == jnp.arange(E, dtype=jnp.int32)[None, :])
    oh_i = onehot.astype(jnp.int32)
    counts = jnp.sum(oh_i, axis=0)
    csum = jnp.cumsum(oh_i, axis=0)
    rank = jnp.take_along_axis(csum, ef[:, None], axis=1)[:, 0] - 1
    tiles_e = (counts + TILE - 1) // TILE
    cum_tiles = jnp.cumsum(tiles_e)
    total_tiles = cum_tiles[E - 1]
    padded_off = (cum_tiles - tiles_e) * TILE
    pos = padded_off[ef] + rank
    gtok, gwf = _sc_scatter(pos, jnp.arange(TK, dtype=jnp.int32) // K, gf)
    tj = jnp.arange(NT, dtype=jnp.int32)
    texp = jnp.searchsorted(
        cum_tiles, jnp.minimum(tj, total_tiles - 1), side="right"
    ).astype(jnp.int32)
    act = (tj < total_tiles).astype(jnp.int32)
    pp0 = pos[0::2]
    pp1 = pos[1::2]
    gw3 = gwf.reshape(NT, 1, TILE)
    return gtok, gw3, texp, act, pp0, pp1


def kernel(x, gate_w, W1, W3, W2, sw1, sw3, sw2):
    gwt = gate_w.T
    s1 = sw1.T
    s3 = sw3.T
    s2 = sw2.T
    eid, g = _router(x, gwt)
    gtok, gw3, texp, act, pp0, pp1 = _routing_metadata(eid, g)
    sh = _shared(x, s1, s3, s2)
    ys = _grouped_gemm(texp, act, gtok, x, W1, W3, W2, gw3)
    return _sc_combine(ys, sh, pp0, pp1)


# speedup vs baseline: 1.0535x; 1.0535x over previous
import functools

import jax
import jax.numpy as jnp
from jax import lax
from jax.experimental import pallas as pl
from jax.experimental.pallas import tpu as pltpu
from jax.experimental.pallas import tpu_sc as plsc

T = 2048
DIM = 1024
INTER = 512
E = 64
K = 2
TK = T * K
TILE = 128
NT = 96
NP = NT * TILE

SC_CORES = 2
SC_SUBCORES = 16
NW = SC_CORES * SC_SUBCORES


def _router_body(x_ref, gwt_ref, eid_ref, g_ref, rank_ref, cnt_ref, counts):
    i = pl.program_id(0)

    @pl.when(i == 0)
    def _():
        counts[...] = jnp.zeros_like(counts)

    xb = x_ref[...]
    logits = jnp.dot(xb, gwt_ref[...], preferred_element_type=jnp.float32)
    scores = jax.nn.sigmoid(logits)
    cols = lax.broadcasted_iota(jnp.int32, scores.shape, 1)
    m1 = jnp.max(scores, axis=1)
    a1 = jnp.argmax(scores, axis=1).astype(jnp.int32)
    masked = jnp.where(cols == a1[:, None], -jnp.inf, scores)
    m2 = jnp.max(masked, axis=1)
    a2 = jnp.argmax(masked, axis=1).astype(jnp.int32)
    s = jnp.maximum(m1 + m2, 1e-12)
    eid_ref[...] = jnp.concatenate([a1[:, None], a2[:, None]], axis=1)
    g_ref[...] = jnp.concatenate([(m1 / s)[:, None], (m2 / s)[:, None]], axis=1)
    bt = xb.shape[0]
    oh = jnp.concatenate([(cols == a1[:, None]).astype(jnp.int32),
                          (cols == a2[:, None]).astype(jnp.int32)], axis=0)
    csum = oh
    sh = 1
    while sh < 2 * bt:
        z = jnp.zeros((sh, E), jnp.int32)
        csum = csum + jnp.concatenate([z, csum[:-sh]], axis=0)
        sh *= 2
    rank_blk = csum - 1 + counts[...]
    r1 = jnp.sum(oh[:bt] * rank_blk[:bt], axis=1)
    r2 = jnp.sum(oh[bt:] * rank_blk[bt:], axis=1)
    rank_ref[...] = jnp.concatenate([r1[:, None], r2[:, None]], axis=1)
    counts[...] = counts[...] + csum[2 * bt - 1 : 2 * bt]
    cnt_ref[...] = counts[...]


def _router(x, gwt):
    bt = 512
    grid = (T // bt,)
    return pl.pallas_call(
        _router_body,
        grid=grid,
        in_specs=[
            pl.BlockSpec((bt, DIM), lambda i: (i, 0)),
            pl.BlockSpec((DIM, E), lambda i: (0, 0)),
        ],
        out_specs=[
            pl.BlockSpec((bt, K), lambda i: (i, 0)),
            pl.BlockSpec((bt, K), lambda i: (i, 0)),
            pl.BlockSpec((bt, K), lambda i: (i, 0)),
            pl.BlockSpec((1, E), lambda i: (0, 0)),
        ],
        out_shape=[
            jax.ShapeDtypeStruct((T, K), jnp.int32),
            jax.ShapeDtypeStruct((T, K), jnp.float32),
            jax.ShapeDtypeStruct((T, K), jnp.int32),
            jax.ShapeDtypeStruct((1, E), jnp.int32),
        ],
        scratch_shapes=[pltpu.VMEM((1, E), jnp.int32)],
    )(x, gwt)


def _shared_body(x_ref, s1_ref, s3_ref, s2_ref, sh_ref):
    xb = x_ref[...]
    h = jax.nn.silu(jnp.dot(xb, s1_ref[...], preferred_element_type=jnp.float32))
    h = h * jnp.dot(xb, s3_ref[...], preferred_element_type=jnp.float32)
    sh_ref[...] = jnp.dot(h, s2_ref[...], preferred_element_type=jnp.float32)


def _shared(x, s1, s3, s2):
    bt = 256
    grid = (T // bt,)
    return pl.pallas_call(
        _shared_body,
        grid=grid,
        in_specs=[
            pl.BlockSpec((bt, DIM), lambda i: (i, 0)),
            pl.BlockSpec((DIM, DIM), lambda i: (0, 0)),
            pl.BlockSpec((DIM, DIM), lambda i: (0, 0)),
            pl.BlockSpec((DIM, DIM), lambda i: (0, 0)),
        ],
        out_specs=pl.BlockSpec((bt, DIM), lambda i: (i, 0)),
        out_shape=jax.ShapeDtypeStruct((T, DIM), jnp.float32),
    )(x, s1, s3, s2)


def _gemm_body(texp_ref, act_ref, gtok_ref, x_any, w1_ref, w3_ref, w2_ref,
               gw_ref, ys_ref, rows, sems):
    j = pl.program_id(0)

    def issue(tj):
        slot = lax.rem(tj, 2)
        base = tj * TILE

        def cp(i, c):
            tok = jnp.clip(gtok_ref[base + i], 0, T - 1)
            pltpu.make_async_copy(
                x_any.at[pl.ds(tok, 1), :],
                rows.at[slot, pl.ds(i, 1), :],
                sems.at[slot],
            ).start()
            return c

        lax.fori_loop(0, TILE, cp, 0)

    @pl.when(j == 0)
    def _():
        issue(0)

    nxt = jnp.minimum(j + 1, NT - 1)

    @pl.when(jnp.logical_and(j + 1 < NT, act_ref[nxt] != 0))
    def _():
        issue(j + 1)

    @pl.when(act_ref[j] != 0)
    def _():
        slot = lax.rem(j, 2)
        pltpu.make_async_copy(
            x_any.at[pl.ds(0, TILE), :], rows.at[slot], sems.at[slot],
        ).wait()
        xb = rows[slot]
        h = jax.nn.silu(jnp.dot(xb, w1_ref[0], preferred_element_type=jnp.float32))
        h = h * jnp.dot(xb, w3_ref[0], preferred_element_type=jnp.float32)
        y = jnp.dot(h, w2_ref[0], preferred_element_type=jnp.float32)
        ys_ref[...] = y * gw_ref[0, 0][:, None]


def _grouped_gemm(texp, act, gtok, x, W1, W3, W2, gw3):
    grid_spec = pltpu.PrefetchScalarGridSpec(
        num_scalar_prefetch=3,
        grid=(NT,),
        in_specs=[
            pl.BlockSpec(memory_space=pl.ANY),
            pl.BlockSpec((1, DIM, INTER), lambda j, texp, act, gtok: (texp[j], 0, 0)),
            pl.BlockSpec((1, DIM, INTER), lambda j, texp, act, gtok: (texp[j], 0, 0)),
            pl.BlockSpec((1, INTER, DIM), lambda j, texp, act, gtok: (texp[j], 0, 0)),
            pl.BlockSpec((1, 1, TILE), lambda j, texp, act, gtok: (j, 0, 0)),
        ],
        out_specs=pl.BlockSpec((TILE, DIM), lambda j, texp, act, gtok: (j, 0)),
        scratch_shapes=[
            pltpu.VMEM((2, TILE, DIM), jnp.float32),
            pltpu.SemaphoreType.DMA((2,)),
        ],
    )
    return pl.pallas_call(
        _gemm_body,
        grid_spec=grid_spec,
        out_shape=jax.ShapeDtypeStruct((NP, DIM), jnp.float32),
    )(texp, act, gtok, x, W1, W3, W2, gw3)


def _sc_scatter(pos, tokv, wv):
    ppw = TK // NW
    mesh = plsc.VectorSubcoreMesh(core_axis_name="c", subcore_axis_name="s")

    @functools.partial(
        pl.kernel,
        mesh=mesh,
        out_type=[
            jax.ShapeDtypeStruct((NP,), jnp.int32),
            jax.ShapeDtypeStruct((NP,), jnp.float32),
        ],
        scratch_types=[
            pltpu.VMEM((ppw,), jnp.int32),
            pltpu.VMEM((ppw,), jnp.int32),
            pltpu.VMEM((ppw,), jnp.float32),
            pltpu.SemaphoreType.DMA,
            pltpu.SemaphoreType.DMA,
        ],
    )
    def k(pos_hbm, tok_hbm, wv_hbm, gtok_hbm, gwf_hbm, idx_v, tv, wvv, s1, s2):
        wid = lax.axis_index("s") * SC_CORES + lax.axis_index("c")
        base = pl.multiple_of(wid * ppw, ppw)
        pltpu.sync_copy(pos_hbm.at[pl.ds(base, ppw)], idx_v)
        pltpu.sync_copy(tok_hbm.at[pl.ds(base, ppw)], tv)
        pltpu.sync_copy(wv_hbm.at[pl.ds(base, ppw)], wvv)
        h1 = pltpu.async_copy(tv, gtok_hbm.at[idx_v], s1)
        h2 = pltpu.async_copy(wvv, gwf_hbm.at[idx_v], s2)
        h1.wait()
        h2.wait()

    return k(pos, tokv, wv)


def _sc_combine(ys, sh, pp0, pp1):
    tok_pw = T // NW
    ch = 32
    mesh = plsc.VectorSubcoreMesh(core_axis_name="c", subcore_axis_name="s")

    @functools.partial(
        pl.kernel,
        mesh=mesh,
        out_type=jax.ShapeDtypeStruct((T, DIM), jnp.float32),
        scratch_types=[
            pltpu.VMEM((ch,), jnp.int32),
            pltpu.VMEM((ch,), jnp.int32),
            pltpu.VMEM((ch, DIM), jnp.float32),
            pltpu.VMEM((ch, DIM), jnp.float32),
            pltpu.VMEM((ch, DIM), jnp.float32),
            pltpu.SemaphoreType.DMA,
        ],
    )
    def k(ys_hbm, sh_hbm, pp0_hbm, pp1_hbm, out_hbm, i0v, i1v, b0, b1, bs,
          sem):
        wid = lax.axis_index("s") * SC_CORES + lax.axis_index("c")
        base = wid * tok_pw

        def chunk(c, carry):
            off = pl.multiple_of(base + c * ch, ch)
            pltpu.sync_copy(pp0_hbm.at[pl.ds(off, ch)], i0v)
            pltpu.sync_copy(pp1_hbm.at[pl.ds(off, ch)], i1v)
            pltpu.async_copy(ys_hbm.at[i0v], b0, sem).wait()
            pltpu.async_copy(ys_hbm.at[i1v], b1, sem).wait()
            pltpu.sync_copy(sh_hbm.at[pl.ds(off, ch)], bs)

            def row(r, rc):
                def col(cc, cc2):
                    sl = pl.ds(pl.multiple_of(cc * 16, 16), 16)
                    b0[r, sl] = b0[r, sl] + b1[r, sl] + bs[r, sl]
                    return cc2
                lax.fori_loop(0, DIM // 16, col, 0)
                return rc

            lax.fori_loop(0, ch, row, 0)
            pltpu.sync_copy(b0, out_hbm.at[pl.ds(off, ch)])
            return carry

        lax.fori_loop(0, tok_pw // ch, chunk, 0)

    return k(ys, sh, pp0, pp1)


def _routing_metadata(eid, g, rank2, cnt):
    ef = eid.reshape(-1)
    gf = g.reshape(-1)
    rank = rank2.reshape(-1)
    counts = cnt[0]
    tiles_e = (counts + TILE - 1) // TILE
    cum_tiles = jnp.cumsum(tiles_e)
    total_tiles = cum_tiles[E - 1]
    padded_off = (cum_tiles - tiles_e) * TILE
    pos = padded_off[ef] + rank
    gtok, gwf = _sc_scatter(pos, jnp.arange(TK, dtype=jnp.int32) // K, gf)
    tj = jnp.arange(NT, dtype=jnp.int32)
    texp = jnp.searchsorted(
        cum_tiles, jnp.minimum(tj, total_tiles - 1), side="right"
    ).astype(jnp.int32)
    act = (tj < total_tiles).astype(jnp.int32)
    pp0 = pos[0::2]
    pp1 = pos[1::2]
    gw3 = gwf.reshape(NT, 1, TILE)
    return gtok, gw3, texp, act, pp0, pp1


def kernel(x, gate_w, W1, W3, W2, sw1, sw3, sw2):
    gwt = gate_w.T
    s1 = sw1.T
    s3 = sw3.T
    s2 = sw2.T
    eid, g, rank2, cnt = _router(x, gwt)
    gtok, gw3, texp, act, pp0, pp1 = _routing_metadata(eid, g, rank2, cnt)
    sh = _shared(x, s1, s3, s2)
    ys = _grouped_gemm(texp, act, gtok, x, W1, W3, W2, gw3)
    return _sc_combine(ys, sh, pp0, pp1)


# speedup vs baseline: 1.4227x; 1.3504x over previous
import functools

import jax
import jax.numpy as jnp
from jax import lax
from jax.experimental import pallas as pl
from jax.experimental.pallas import tpu as pltpu
from jax.experimental.pallas import tpu_sc as plsc

T = 2048
DIM = 1024
INTER = 512
E = 64
K = 2
TK = T * K
TILE = 128
NT = 96
NP = NT * TILE

SC_CORES = 2
SC_SUBCORES = 16
NW = SC_CORES * SC_SUBCORES


def _router_body(x_ref, gwt_ref, eid_ref, g_ref, rank_ref, cnt_ref, counts):
    i = pl.program_id(0)

    @pl.when(i == 0)
    def _():
        counts[...] = jnp.zeros_like(counts)

    xb = x_ref[...]
    logits = jnp.dot(xb, gwt_ref[...], preferred_element_type=jnp.float32)
    scores = jax.nn.sigmoid(logits)
    cols = lax.broadcasted_iota(jnp.int32, scores.shape, 1)
    m1 = jnp.max(scores, axis=1)
    a1 = jnp.argmax(scores, axis=1).astype(jnp.int32)
    masked = jnp.where(cols == a1[:, None], -jnp.inf, scores)
    m2 = jnp.max(masked, axis=1)
    a2 = jnp.argmax(masked, axis=1).astype(jnp.int32)
    s = jnp.maximum(m1 + m2, 1e-12)
    eid_ref[...] = jnp.concatenate([a1[:, None], a2[:, None]], axis=1)
    g_ref[...] = jnp.concatenate([(m1 / s)[:, None], (m2 / s)[:, None]], axis=1)
    bt = xb.shape[0]
    oh = jnp.concatenate([(cols == a1[:, None]).astype(jnp.int32),
                          (cols == a2[:, None]).astype(jnp.int32)], axis=0)
    csum = oh
    sh = 1
    while sh < 2 * bt:
        z = jnp.zeros((sh, E), jnp.int32)
        csum = csum + jnp.concatenate([z, csum[:-sh]], axis=0)
        sh *= 2
    rank_blk = csum - 1 + counts[...]
    r1 = jnp.sum(oh[:bt] * rank_blk[:bt], axis=1)
    r2 = jnp.sum(oh[bt:] * rank_blk[bt:], axis=1)
    rank_ref[...] = jnp.concatenate([r1[:, None], r2[:, None]], axis=1)
    counts[...] = counts[...] + csum[2 * bt - 1 : 2 * bt]
    cnt_ref[...] = counts[...]


def _router(x, gwt):
    bt = 512
    grid = (T // bt,)
    return pl.pallas_call(
        _router_body,
        grid=grid,
        in_specs=[
            pl.BlockSpec((bt, DIM), lambda i: (i, 0)),
            pl.BlockSpec((DIM, E), lambda i: (0, 0)),
        ],
        out_specs=[
            pl.BlockSpec((bt, K), lambda i: (i, 0)),
            pl.BlockSpec((bt, K), lambda i: (i, 0)),
            pl.BlockSpec((bt, K), lambda i: (i, 0)),
            pl.BlockSpec((1, E), lambda i: (0, 0)),
        ],
        out_shape=[
            jax.ShapeDtypeStruct((T, K), jnp.int32),
            jax.ShapeDtypeStruct((T, K), jnp.float32),
            jax.ShapeDtypeStruct((T, K), jnp.int32),
            jax.ShapeDtypeStruct((1, E), jnp.int32),
        ],
        scratch_shapes=[pltpu.VMEM((1, E), jnp.int32)],
    )(x, gwt)


def _shared_body(x_ref, s1_ref, s3_ref, s2_ref, sh_ref):
    xb = x_ref[...]
    h = jax.nn.silu(jnp.dot(xb, s1_ref[...], preferred_element_type=jnp.float32))
    h = h * jnp.dot(xb, s3_ref[...], preferred_element_type=jnp.float32)
    sh_ref[...] = jnp.dot(h, s2_ref[...], preferred_element_type=jnp.float32)


def _shared(x, s1, s3, s2):
    bt = 256
    grid = (T // bt,)
    return pl.pallas_call(
        _shared_body,
        grid=grid,
        in_specs=[
            pl.BlockSpec((bt, DIM), lambda i: (i, 0)),
            pl.BlockSpec((DIM, DIM), lambda i: (0, 0)),
            pl.BlockSpec((DIM, DIM), lambda i: (0, 0)),
            pl.BlockSpec((DIM, DIM), lambda i: (0, 0)),
        ],
        out_specs=pl.BlockSpec((bt, DIM), lambda i: (i, 0)),
        out_shape=jax.ShapeDtypeStruct((T, DIM), jnp.float32),
    )(x, s1, s3, s2)


def _gemm_body(texp_ref, act_ref, gtok_ref, tlen_ref, oidx_ref, x_any,
               w1_ref, w3_ref, w2_ref, gw_ref, ys_ref, rows, sems):
    j = pl.program_id(0)

    def issue(tj):
        slot = lax.rem(tj, 2)
        base = tj * TILE

        def cp(i, c):
            tok = jnp.clip(gtok_ref[base + i], 0, T - 1)
            pltpu.make_async_copy(
                x_any.at[pl.ds(tok, 1), :],
                rows.at[slot, pl.ds(i, 1), :],
                sems.at[slot],
            ).start()
            return c

        lax.fori_loop(0, tlen_ref[tj], cp, 0)

    @pl.when(j == 0)
    def _():
        issue(0)

    @pl.when(j + 1 < NT)
    def _():
        issue(jnp.minimum(j + 1, NT - 1))

    @pl.when(act_ref[j] != 0)
    def _():
        slot = lax.rem(j, 2)

        def drain(i, c):
            pltpu.make_async_copy(
                x_any.at[pl.ds(0, 1), :], rows.at[slot, pl.ds(0, 1), :],
                sems.at[slot],
            ).wait()
            return c

        lax.fori_loop(0, tlen_ref[j], drain, 0)
        xb = rows[slot]
        h = jax.nn.silu(jnp.dot(xb, w1_ref[0], preferred_element_type=jnp.float32))
        h = h * jnp.dot(xb, w3_ref[0], preferred_element_type=jnp.float32)
        y = jnp.dot(h, w2_ref[0], preferred_element_type=jnp.float32)
        ys_ref[...] = y * gw_ref[0, 0][:, None]


def _grouped_gemm(texp, act, gtok, tlen, oidx, x, W1, W3, W2, gw3):
    grid_spec = pltpu.PrefetchScalarGridSpec(
        num_scalar_prefetch=5,
        grid=(NT,),
        in_specs=[
            pl.BlockSpec(memory_space=pl.ANY),
            pl.BlockSpec((1, DIM, INTER), lambda j, texp, act, gtok, tlen, oidx: (texp[j], 0, 0)),
            pl.BlockSpec((1, DIM, INTER), lambda j, texp, act, gtok, tlen, oidx: (texp[j], 0, 0)),
            pl.BlockSpec((1, INTER, DIM), lambda j, texp, act, gtok, tlen, oidx: (texp[j], 0, 0)),
            pl.BlockSpec((1, 1, TILE), lambda j, texp, act, gtok, tlen, oidx: (j, 0, 0)),
        ],
        out_specs=pl.BlockSpec((TILE, DIM), lambda j, texp, act, gtok, tlen, oidx: (oidx[j], 0)),
        scratch_shapes=[
            pltpu.VMEM((2, TILE, DIM), jnp.float32),
            pltpu.SemaphoreType.DMA((2,)),
        ],
    )
    return pl.pallas_call(
        _gemm_body,
        grid_spec=grid_spec,
        out_shape=jax.ShapeDtypeStruct((NP, DIM), jnp.float32),
    )(texp, act, gtok, tlen, oidx, x, W1, W3, W2, gw3)


def _sc_scatter(pos, tokv, wv):
    ppw = TK // NW
    mesh = plsc.VectorSubcoreMesh(core_axis_name="c", subcore_axis_name="s")

    @functools.partial(
        pl.kernel,
        mesh=mesh,
        out_type=[
            jax.ShapeDtypeStruct((NP,), jnp.int32),
            jax.ShapeDtypeStruct((NP,), jnp.float32),
        ],
        scratch_types=[
            pltpu.VMEM((ppw,), jnp.int32),
            pltpu.VMEM((ppw,), jnp.int32),
            pltpu.VMEM((ppw,), jnp.float32),
            pltpu.SemaphoreType.DMA,
            pltpu.SemaphoreType.DMA,
        ],
    )
    def k(pos_hbm, tok_hbm, wv_hbm, gtok_hbm, gwf_hbm, idx_v, tv, wvv, s1, s2):
        wid = lax.axis_index("s") * SC_CORES + lax.axis_index("c")
        base = pl.multiple_of(wid * ppw, ppw)
        pltpu.sync_copy(pos_hbm.at[pl.ds(base, ppw)], idx_v)
        pltpu.sync_copy(tok_hbm.at[pl.ds(base, ppw)], tv)
        pltpu.sync_copy(wv_hbm.at[pl.ds(base, ppw)], wvv)
        h1 = pltpu.async_copy(tv, gtok_hbm.at[idx_v], s1)
        h2 = pltpu.async_copy(wvv, gwf_hbm.at[idx_v], s2)
        h1.wait()
        h2.wait()

    return k(pos, tokv, wv)


def _sc_combine(ys, sh, pp0, pp1):
    tok_pw = T // NW
    ch = 32
    mesh = plsc.VectorSubcoreMesh(core_axis_name="c", subcore_axis_name="s")

    @functools.partial(
        pl.kernel,
        mesh=mesh,
        out_type=jax.ShapeDtypeStruct((T, DIM), jnp.float32),
        scratch_types=[
            pltpu.VMEM((ch,), jnp.int32),
            pltpu.VMEM((ch,), jnp.int32),
            pltpu.VMEM((ch, DIM), jnp.float32),
            pltpu.VMEM((ch, DIM), jnp.float32),
            pltpu.VMEM((ch, DIM), jnp.float32),
            pltpu.SemaphoreType.DMA,
        ],
    )
    def k(ys_hbm, sh_hbm, pp0_hbm, pp1_hbm, out_hbm, i0v, i1v, b0, b1, bs,
          sem):
        wid = lax.axis_index("s") * SC_CORES + lax.axis_index("c")
        base = wid * tok_pw

        def chunk(c, carry):
            off = pl.multiple_of(base + c * ch, ch)
            pltpu.sync_copy(pp0_hbm.at[pl.ds(off, ch)], i0v)
            pltpu.sync_copy(pp1_hbm.at[pl.ds(off, ch)], i1v)
            pltpu.async_copy(ys_hbm.at[i0v], b0, sem).wait()
            pltpu.async_copy(ys_hbm.at[i1v], b1, sem).wait()
            pltpu.sync_copy(sh_hbm.at[pl.ds(off, ch)], bs)

            def row(r, rc):
                def col(cc, cc2):
                    sl = pl.ds(pl.multiple_of(cc * 16, 16), 16)
                    b0[r, sl] = b0[r, sl] + b1[r, sl] + bs[r, sl]
                    return cc2
                lax.fori_loop(0, DIM // 16, col, 0)
                return rc

            lax.fori_loop(0, ch, row, 0)
            pltpu.sync_copy(b0, out_hbm.at[pl.ds(off, ch)])
            return carry

        lax.fori_loop(0, tok_pw // ch, chunk, 0)

    return k(ys, sh, pp0, pp1)


def _routing_metadata(eid, g, rank2, cnt):
    ef = eid.reshape(-1)
    gf = g.reshape(-1)
    rank = rank2.reshape(-1)
    counts = cnt[0]
    tiles_e = (counts + TILE - 1) // TILE
    cum_tiles = jnp.cumsum(tiles_e)
    total_tiles = cum_tiles[E - 1]
    padded_off = (cum_tiles - tiles_e) * TILE
    pos = padded_off[ef] + rank
    gtok, gwf = _sc_scatter(pos, jnp.arange(TK, dtype=jnp.int32) // K, gf)
    tj = jnp.arange(NT, dtype=jnp.int32)
    tjc = jnp.minimum(tj, total_tiles - 1)
    texp = jnp.sum((cum_tiles[None, :] <= tjc[:, None]).astype(jnp.int32),
                   axis=1)
    act = (tj < total_tiles).astype(jnp.int32)
    first_tile = (cum_tiles - tiles_e)[texp]
    tlen = jnp.clip(counts[texp] - (tjc - first_tile) * TILE, 0, TILE) * act
    oidx = jnp.where(act == 1, tj, total_tiles - 1)
    pp0 = pos[0::2]
    pp1 = pos[1::2]
    gw3 = gwf.reshape(NT, 1, TILE)
    return gtok, gw3, texp, act, tlen, oidx, pp0, pp1


def kernel(x, gate_w, W1, W3, W2, sw1, sw3, sw2):
    gwt = gate_w.T
    s1 = sw1.T
    s3 = sw3.T
    s2 = sw2.T
    eid, g, rank2, cnt = _router(x, gwt)
    gtok, gw3, texp, act, tlen, oidx, pp0, pp1 = _routing_metadata(
        eid, g, rank2, cnt)
    sh = _shared(x, s1, s3, s2)
    ys = _grouped_gemm(texp, act, gtok, tlen, oidx, x, W1, W3, W2, gw3)
    return _sc_combine(ys, sh, pp0, pp1)


# speedup vs baseline: 1.4641x; 1.0291x over previous
import functools

import jax
import jax.numpy as jnp
from jax import lax
from jax.experimental import pallas as pl
from jax.experimental.pallas import tpu as pltpu
from jax.experimental.pallas import tpu_sc as plsc

T = 2048
DIM = 1024
INTER = 512
E = 64
K = 2
TK = T * K
TILE = 128
NT = 96
NP = NT * TILE

SC_CORES = 2
SC_SUBCORES = 16
NW = SC_CORES * SC_SUBCORES


def _router_body(x_ref, gwt_ref, eid_ref, g_ref, rank_ref, cnt_ref, counts):
    i = pl.program_id(0)

    @pl.when(i == 0)
    def _():
        counts[...] = jnp.zeros_like(counts)

    xb = x_ref[...]
    logits = jnp.dot(xb, gwt_ref[...], preferred_element_type=jnp.float32)
    scores = jax.nn.sigmoid(logits)
    cols = lax.broadcasted_iota(jnp.int32, scores.shape, 1)
    m1 = jnp.max(scores, axis=1)
    a1 = jnp.argmax(scores, axis=1).astype(jnp.int32)
    masked = jnp.where(cols == a1[:, None], -jnp.inf, scores)
    m2 = jnp.max(masked, axis=1)
    a2 = jnp.argmax(masked, axis=1).astype(jnp.int32)
    s = jnp.maximum(m1 + m2, 1e-12)
    eid_ref[...] = jnp.concatenate([a1[:, None], a2[:, None]], axis=1)
    g_ref[...] = jnp.concatenate([(m1 / s)[:, None], (m2 / s)[:, None]], axis=1)
    bt = xb.shape[0]
    oh = jnp.concatenate([(cols == a1[:, None]).astype(jnp.int32),
                          (cols == a2[:, None]).astype(jnp.int32)], axis=0)
    csum = oh
    sh = 1
    while sh < 2 * bt:
        z = jnp.zeros((sh, E), jnp.int32)
        csum = csum + jnp.concatenate([z, csum[:-sh]], axis=0)
        sh *= 2
    rank_blk = csum - 1 + counts[...]
    r1 = jnp.sum(oh[:bt] * rank_blk[:bt], axis=1)
    r2 = jnp.sum(oh[bt:] * rank_blk[bt:], axis=1)
    rank_ref[...] = jnp.concatenate([r1[:, None], r2[:, None]], axis=1)
    counts[...] = counts[...] + csum[2 * bt - 1 : 2 * bt]
    cnt_ref[...] = counts[...]


def _router(x, gwt):
    bt = 512
    grid = (T // bt,)
    return pl.pallas_call(
        _router_body,
        grid=grid,
        in_specs=[
            pl.BlockSpec((bt, DIM), lambda i: (i, 0)),
            pl.BlockSpec((DIM, E), lambda i: (0, 0)),
        ],
        out_specs=[
            pl.BlockSpec((bt, K), lambda i: (i, 0)),
            pl.BlockSpec((bt, K), lambda i: (i, 0)),
            pl.BlockSpec((bt, K), lambda i: (i, 0)),
            pl.BlockSpec((1, E), lambda i: (0, 0)),
        ],
        out_shape=[
            jax.ShapeDtypeStruct((T, K), jnp.int32),
            jax.ShapeDtypeStruct((T, K), jnp.float32),
            jax.ShapeDtypeStruct((T, K), jnp.int32),
            jax.ShapeDtypeStruct((1, E), jnp.int32),
        ],
        scratch_shapes=[pltpu.VMEM((1, E), jnp.int32)],
    )(x, gwt)


def _shared_body(x_ref, s1_ref, s3_ref, s2_ref, sh_ref):
    xb = x_ref[...]
    h = jax.nn.silu(jnp.dot(xb, s1_ref[...], preferred_element_type=jnp.float32))
    h = h * jnp.dot(xb, s3_ref[...], preferred_element_type=jnp.float32)
    sh_ref[...] = jnp.dot(h, s2_ref[...], preferred_element_type=jnp.float32)


def _shared(x, s1, s3, s2):
    bt = 256
    grid = (T // bt,)
    return pl.pallas_call(
        _shared_body,
        grid=grid,
        in_specs=[
            pl.BlockSpec((bt, DIM), lambda i: (i, 0)),
            pl.BlockSpec((DIM, DIM), lambda i: (0, 0)),
            pl.BlockSpec((DIM, DIM), lambda i: (0, 0)),
            pl.BlockSpec((DIM, DIM), lambda i: (0, 0)),
        ],
        out_specs=pl.BlockSpec((bt, DIM), lambda i: (i, 0)),
        out_shape=jax.ShapeDtypeStruct((T, DIM), jnp.float32),
    )(x, s1, s3, s2)


def _gemm_body(texp_ref, act_ref, gtok_ref, tlen_ref, oidx_ref, x_any,
               w1_ref, w3_ref, w2_ref, gw_ref, ys_ref, rows, sems):
    j = pl.program_id(0)

    def issue(tj):
        slot = lax.rem(tj, 2)
        base = tj * TILE

        def cp(i, c):
            tok = jnp.clip(gtok_ref[base + i], 0, T - 1)
            pltpu.make_async_copy(
                x_any.at[pl.ds(tok, 1), :],
                rows.at[slot, pl.ds(i, 1), :],
                sems.at[slot],
            ).start()
            return c

        lax.fori_loop(0, tlen_ref[tj], cp, 0)

    @pl.when(j == 0)
    def _():
        issue(0)

    @pl.when(j + 1 < NT)
    def _():
        issue(jnp.minimum(j + 1, NT - 1))

    @pl.when(act_ref[j] != 0)
    def _():
        slot = lax.rem(j, 2)

        def drain(i, c):
            pltpu.make_async_copy(
                x_any.at[pl.ds(0, 8), :], rows.at[slot, pl.ds(0, 8), :],
                sems.at[slot],
            ).wait()
            return c

        lax.fori_loop(0, tlen_ref[j] // 8, drain, 0)
        xb = rows[slot]
        h = jax.nn.silu(jnp.dot(xb, w1_ref[0], preferred_element_type=jnp.float32))
        h = h * jnp.dot(xb, w3_ref[0], preferred_element_type=jnp.float32)
        y = jnp.dot(h, w2_ref[0], preferred_element_type=jnp.float32)
        ys_ref[...] = y * gw_ref[0, 0][:, None]


def _grouped_gemm(texp, act, gtok, tlen, oidx, x, W1, W3, W2, gw3):
    grid_spec = pltpu.PrefetchScalarGridSpec(
        num_scalar_prefetch=5,
        grid=(NT,),
        in_specs=[
            pl.BlockSpec(memory_space=pl.ANY),
            pl.BlockSpec((1, DIM, INTER), lambda j, texp, act, gtok, tlen, oidx: (texp[j], 0, 0)),
            pl.BlockSpec((1, DIM, INTER), lambda j, texp, act, gtok, tlen, oidx: (texp[j], 0, 0)),
            pl.BlockSpec((1, INTER, DIM), lambda j, texp, act, gtok, tlen, oidx: (texp[j], 0, 0)),
            pl.BlockSpec((1, 1, TILE), lambda j, texp, act, gtok, tlen, oidx: (j, 0, 0)),
        ],
        out_specs=pl.BlockSpec((TILE, DIM), lambda j, texp, act, gtok, tlen, oidx: (oidx[j], 0)),
        scratch_shapes=[
            pltpu.VMEM((2, TILE, DIM), jnp.float32),
            pltpu.SemaphoreType.DMA((2,)),
        ],
    )
    return pl.pallas_call(
        _gemm_body,
        grid_spec=grid_spec,
        out_shape=jax.ShapeDtypeStruct((NP, DIM), jnp.float32),
    )(texp, act, gtok, tlen, oidx, x, W1, W3, W2, gw3)


def _sc_scatter(pos, tokv, wv):
    ppw = TK // NW
    mesh = plsc.VectorSubcoreMesh(core_axis_name="c", subcore_axis_name="s")

    @functools.partial(
        pl.kernel,
        mesh=mesh,
        out_type=[
            jax.ShapeDtypeStruct((NP,), jnp.int32),
            jax.ShapeDtypeStruct((NP,), jnp.float32),
        ],
        scratch_types=[
            pltpu.VMEM((ppw,), jnp.int32),
            pltpu.VMEM((ppw,), jnp.int32),
            pltpu.VMEM((ppw,), jnp.float32),
            pltpu.SemaphoreType.DMA,
            pltpu.SemaphoreType.DMA,
        ],
    )
    def k(pos_hbm, tok_hbm, wv_hbm, gtok_hbm, gwf_hbm, idx_v, tv, wvv, s1, s2):
        wid = lax.axis_index("s") * SC_CORES + lax.axis_index("c")
        base = pl.multiple_of(wid * ppw, ppw)
        pltpu.sync_copy(pos_hbm.at[pl.ds(base, ppw)], idx_v)
        pltpu.sync_copy(tok_hbm.at[pl.ds(base, ppw)], tv)
        pltpu.sync_copy(wv_hbm.at[pl.ds(base, ppw)], wvv)
        h1 = pltpu.async_copy(tv, gtok_hbm.at[idx_v], s1)
        h2 = pltpu.async_copy(wvv, gwf_hbm.at[idx_v], s2)
        h1.wait()
        h2.wait()

    return k(pos, tokv, wv)


def _sc_combine(ys, sh, pp0, pp1):
    tok_pw = T // NW
    ch = 32
    mesh = plsc.VectorSubcoreMesh(core_axis_name="c", subcore_axis_name="s")

    @functools.partial(
        pl.kernel,
        mesh=mesh,
        out_type=jax.ShapeDtypeStruct((T, DIM), jnp.float32),
        scratch_types=[
            pltpu.VMEM((ch,), jnp.int32),
            pltpu.VMEM((ch,), jnp.int32),
            pltpu.VMEM((ch, DIM), jnp.float32),
            pltpu.VMEM((ch, DIM), jnp.float32),
            pltpu.VMEM((ch, DIM), jnp.float32),
            pltpu.SemaphoreType.DMA,
        ],
    )
    def k(ys_hbm, sh_hbm, pp0_hbm, pp1_hbm, out_hbm, i0v, i1v, b0, b1, bs,
          sem):
        wid = lax.axis_index("s") * SC_CORES + lax.axis_index("c")
        base = wid * tok_pw

        def chunk(c, carry):
            off = pl.multiple_of(base + c * ch, ch)
            pltpu.sync_copy(pp0_hbm.at[pl.ds(off, ch)], i0v)
            pltpu.sync_copy(pp1_hbm.at[pl.ds(off, ch)], i1v)
            pltpu.async_copy(ys_hbm.at[i0v], b0, sem).wait()
            pltpu.async_copy(ys_hbm.at[i1v], b1, sem).wait()
            pltpu.sync_copy(sh_hbm.at[pl.ds(off, ch)], bs)

            def row(r, rc):
                def col(cc, cc2):
                    sl = pl.ds(pl.multiple_of(cc * 16, 16), 16)
                    b0[r, sl] = b0[r, sl] + b1[r, sl] + bs[r, sl]
                    return cc2
                lax.fori_loop(0, DIM // 16, col, 0)
                return rc

            lax.fori_loop(0, ch, row, 0)
            pltpu.sync_copy(b0, out_hbm.at[pl.ds(off, ch)])
            return carry

        lax.fori_loop(0, tok_pw // ch, chunk, 0)

    return k(ys, sh, pp0, pp1)


def _routing_metadata(eid, g, rank2, cnt):
    ef = eid.reshape(-1)
    gf = g.reshape(-1)
    rank = rank2.reshape(-1)
    counts = cnt[0]
    tiles_e = (counts + TILE - 1) // TILE
    cum_tiles = jnp.cumsum(tiles_e)
    total_tiles = cum_tiles[E - 1]
    padded_off = (cum_tiles - tiles_e) * TILE
    pos = padded_off[ef] + rank
    gtok, gwf = _sc_scatter(pos, jnp.arange(TK, dtype=jnp.int32) // K, gf)
    tj = jnp.arange(NT, dtype=jnp.int32)
    tjc = jnp.minimum(tj, total_tiles - 1)
    texp = jnp.sum((cum_tiles[None, :] <= tjc[:, None]).astype(jnp.int32),
                   axis=1)
    act = (tj < total_tiles).astype(jnp.int32)
    first_tile = (cum_tiles - tiles_e)[texp]
    tlen = jnp.clip(counts[texp] - (tjc - first_tile) * TILE, 0, TILE) * act
    tlen = jnp.minimum(((tlen + 7) // 8) * 8, TILE)
    oidx = jnp.where(act == 1, tj, total_tiles - 1)
    pp0 = pos[0::2]
    pp1 = pos[1::2]
    gw3 = gwf.reshape(NT, 1, TILE)
    return gtok, gw3, texp, act, tlen, oidx, pp0, pp1


def kernel(x, gate_w, W1, W3, W2, sw1, sw3, sw2):
    gwt = gate_w.T
    s1 = sw1.T
    s3 = sw3.T
    s2 = sw2.T
    eid, g, rank2, cnt = _router(x, gwt)
    gtok, gw3, texp, act, tlen, oidx, pp0, pp1 = _routing_metadata(
        eid, g, rank2, cnt)
    sh = _shared(x, s1, s3, s2)
    ys = _grouped_gemm(texp, act, gtok, tlen, oidx, x, W1, W3, W2, gw3)
    return _sc_combine(ys, sh, pp0, pp1)
